# baseline (speedup 1.0000x reference)
import functools

import jax
import jax.numpy as jnp
from jax import lax
from jax.experimental import pallas as pl
from jax.experimental.pallas import tpu as pltpu
from jax.experimental.pallas import tpu_sc as plsc

_D = 64
_CHUNK = 128
_K = 4
_NC = 2
_NS = 16
_NW = _NC * _NS


@jax.jit
def _sc_gather(idx2d, vocab):
    n_chunks = idx2d.shape[0]
    rows_per_w = n_chunks // _NW
    n_groups = rows_per_w // _K

    mesh = plsc.VectorSubcoreMesh(core_axis_name="c", subcore_axis_name="s")

    @functools.partial(
        pl.kernel,
        mesh=mesh,
        out_type=jax.ShapeDtypeStruct((n_chunks, _CHUNK, _D), jnp.float32),
        scratch_types=[
            pltpu.VMEM((_K, _CHUNK), jnp.int32),
            pltpu.VMEM((_K, _CHUNK, _D), jnp.float32),
            pltpu.SemaphoreType.DMA,
        ],
        compiler_params=pltpu.CompilerParams(use_tc_tiling_on_sc=False),
    )
    def body(idx_hbm, table_hbm, out_hbm, idx_v, rows_v, sem):
        wid = lax.axis_index("s") * _NC + lax.axis_index("c")
        row0 = wid * rows_per_w

        def group(g, carry):
            gbase = row0 + g * _K
            pltpu.sync_copy(idx_hbm.at[pl.ds(gbase, _K)], idx_v)
            copies = [
                pltpu.async_copy(table_hbm.at[idx_v.at[j]], rows_v.at[j], sem)
                for j in range(_K)
            ]
            for c in copies:
                c.wait()
            pltpu.sync_copy(rows_v, out_hbm.at[pl.ds(gbase, _K)])
            return carry

        lax.fori_loop(0, n_groups, group, 0)

    return body(idx2d, vocab)


def kernel(s, vocab):
    b, t = s.shape
    n = b * t
    idx2d = s.reshape(n // _CHUNK, _CHUNK).astype(jnp.int32)
    out = _sc_gather(idx2d, vocab)
    return out.reshape(b, t, _D)


# speedup vs baseline: 1.0413x; 1.0413x over previous
import functools

import jax
import jax.numpy as jnp
from jax import lax
from jax.experimental import pallas as pl
from jax.experimental.pallas import tpu as pltpu
from jax.experimental.pallas import tpu_sc as plsc

_D = 64
_CHUNK = 128
_K = 5
_NBUF = 2
_NC = 2
_NS = 16
_NW = _NC * _NS


@jax.jit
def _sc_gather(idx2d, vocab):
    n_chunks = idx2d.shape[0]
    rows_per_w = n_chunks // _NW
    n_outer = rows_per_w // (_K * _NBUF)

    mesh = plsc.VectorSubcoreMesh(core_axis_name="c", subcore_axis_name="s")

    @functools.partial(
        pl.kernel,
        mesh=mesh,
        out_type=jax.ShapeDtypeStruct((n_chunks, _CHUNK, _D), jnp.float32),
        scratch_types=[
            pltpu.VMEM((rows_per_w, _CHUNK), jnp.int32),
            pltpu.VMEM((_NBUF, _K, _CHUNK, _D), jnp.float32),
            pltpu.SemaphoreType.DMA,
            pltpu.SemaphoreType.DMA,
            pltpu.SemaphoreType.DMA,
        ],
        compiler_params=pltpu.CompilerParams(use_tc_tiling_on_sc=False),
    )
    def body(idx_hbm, table_hbm, out_hbm, idx_v, rows_v, gsem, wsem0, wsem1):
        wid = lax.axis_index("s") * _NC + lax.axis_index("c")
        row0 = wid * rows_per_w
        wsems = (wsem0, wsem1)

        pltpu.sync_copy(idx_hbm.at[pl.ds(row0, rows_per_w)], idx_v)

        def outer(i, carry):
            for b in range(_NBUF):
                g = i * _NBUF + b
                lbase = g * _K
                gbase = row0 + lbase

                @pl.when(i > 0)
                def _():
                    pltpu.make_async_copy(
                        rows_v.at[b], out_hbm.at[pl.ds(gbase, _K)], wsems[b]
                    ).wait()

                copies = [
                    pltpu.async_copy(
                        table_hbm.at[idx_v.at[lbase + j]], rows_v.at[b, j], gsem
                    )
                    for j in range(_K)
                ]
                for c in copies:
                    c.wait()
                pltpu.async_copy(rows_v.at[b], out_hbm.at[pl.ds(gbase, _K)], wsems[b])
            return carry

        lax.fori_loop(0, n_outer, outer, 0)

        for b in range(_NBUF):
            pltpu.make_async_copy(
                rows_v.at[b], out_hbm.at[pl.ds(row0, _K)], wsems[b]
            ).wait()

    return body(idx2d, vocab)


def kernel(s, vocab):
    b, t = s.shape
    n = b * t
    idx2d = s.reshape(n // _CHUNK, _CHUNK).astype(jnp.int32)
    out = _sc_gather(idx2d, vocab)
    return out.reshape(b, t, _D)
